# baseline (speedup 1.0000x reference)
import jax
import jax.numpy as jnp
from jax import lax
from jax.experimental import pallas as pl
from jax.experimental.pallas import tpu as pltpu

_B = 32
_V = 1000000
_CHUNK = 8192

_K0 = 0
_K1 = 42
_K2 = _K0 ^ _K1 ^ 0x1BD11BDA

_ROT_A = (13, 15, 26, 6)
_ROT_B = (17, 29, 16, 24)


def _rotl(x, r):
    return lax.shift_left(x, jnp.int32(r)) | lax.shift_right_logical(
        x, jnp.int32(32 - r)
    )


def _qround(x0, x1, rots):
    for r in rots:
        x0 = x0 + x1
        x1 = _rotl(x1, r) ^ x0
    return x0, x1


def _threefry_bits(counts):
    x0 = jnp.zeros_like(counts) + jnp.int32(_K0)
    x1 = counts + jnp.int32(_K1)
    x0, x1 = _qround(x0, x1, _ROT_A)
    x0, x1 = x0 + jnp.int32(_K1), x1 + jnp.int32(_K2 + 1)
    x0, x1 = _qround(x0, x1, _ROT_B)
    x0, x1 = x0 + jnp.int32(_K2), x1 + jnp.int32(_K0 + 2)
    x0, x1 = _qround(x0, x1, _ROT_A)
    x0, x1 = x0 + jnp.int32(_K0), x1 + jnp.int32(_K1 + 3)
    x0, x1 = _qround(x0, x1, _ROT_B)
    x0, x1 = x0 + jnp.int32(_K1), x1 + jnp.int32(_K2 + 4)
    x0, x1 = _qround(x0, x1, _ROT_A)
    x0, x1 = x0 + jnp.int32(_K2), x1 + jnp.int32(_K0 + 5)
    return x0 ^ x1


def _sample_block(logits_ref, val_ref, idx_ref):
    step = pl.program_id(0)
    col = lax.broadcasted_iota(jnp.int32, (_B, _CHUNK), 1) + step * jnp.int32(_CHUNK)
    row = lax.broadcasted_iota(jnp.int32, (_B, _CHUNK), 0)
    counts = row * jnp.int32(_V) + col
    bits = _threefry_bits(counts)
    fb = lax.shift_right_logical(bits, jnp.int32(9)) | jnp.int32(0x3F800000)
    u = lax.bitcast_convert_type(fb, jnp.float32) - jnp.float32(1.0)
    u = jnp.where(u == 0.0, jnp.float32(1e-20), u)
    g = -jnp.log(-jnp.log(u))
    val = logits_ref[...] + g
    val = jnp.where(col < jnp.int32(_V), val, -jnp.inf)
    m = jnp.max(val, axis=1)
    idx = jnp.min(
        jnp.where(val == m[:, None], col, jnp.int32(2147483647)), axis=1
    )
    val_ref[0, 0, :] = m
    idx_ref[0, 0, :] = idx


def kernel(logits):
    nblk = pl.cdiv(_V, _CHUNK)
    vals, idxs = pl.pallas_call(
        _sample_block,
        grid=(nblk,),
        in_specs=[pl.BlockSpec((_B, _CHUNK), lambda i: (0, i))],
        out_specs=[
            pl.BlockSpec((1, 1, _B), lambda i: (i, 0, 0)),
            pl.BlockSpec((1, 1, _B), lambda i: (i, 0, 0)),
        ],
        out_shape=[
            jax.ShapeDtypeStruct((nblk, 1, _B), jnp.float32),
            jax.ShapeDtypeStruct((nblk, 1, _B), jnp.int32),
        ],
        compiler_params=pltpu.CompilerParams(
            dimension_semantics=("parallel",),
        ),
    )(logits)
    vals = vals.reshape(nblk, _B)
    idxs = idxs.reshape(nblk, _B)
    best_blk = jnp.argmax(vals, axis=0)
    return jnp.take_along_axis(idxs, best_blk[None, :], axis=0)[0]
